# Initial kernel scaffold; baseline (speedup 1.0000x reference)
import jax
import jax.numpy as jnp
from jax.experimental import pallas as pl


def kernel(x, S0, W_kvqm, alpha):
    raise NotImplementedError("write your pallas kernel here")


# speedup vs baseline: 3.8440x; 3.8440x over previous
import jax
import jax.numpy as jnp
from jax.experimental import pallas as pl
from jax.experimental.pallas import tpu as pltpu

_DIM = 1024
_N = 64
_T_CHUNK = 128
_B_BLK = 16
_EPS_NORM = 1e-6


def _cell_kernel(alpha_ref, x_ref, w_ref, s0_ref, out_ref, sf_ref,
                 k_s, v_s, q_s, m_s):
    t_idx = pl.program_id(1)
    alpha = alpha_ref[0]

    xc = x_ref[...].reshape(_T_CHUNK * _B_BLK, _DIM)
    proj = jnp.dot(xc, w_ref[...], preferred_element_type=jnp.float32)
    k = proj[:, :_N]
    v = proj[:, _N:2 * _N]
    q = proj[:, 2 * _N:3 * _N]
    m = proj[:, 3 * _N:]
    k = k / (jnp.sqrt(jnp.sum(k * k, axis=-1, keepdims=True)) + _EPS_NORM)
    m = m / (jnp.sqrt(jnp.sum(m * m, axis=-1, keepdims=True)) + _EPS_NORM)
    k_s[...] = k.reshape(_T_CHUNK, _B_BLK, _N)
    v_s[...] = v.reshape(_T_CHUNK, _B_BLK, _N)
    q_s[...] = q.reshape(_T_CHUNK, _B_BLK, _N)
    m_s[...] = m.reshape(_T_CHUNK, _B_BLK, _N)

    @pl.when(t_idx == 0)
    def _():
        sf_ref[...] = s0_ref[...]

    def body(t, S):
        k = k_s[t]
        v = v_s[t]
        q = q_s[t]
        m = m_s[t]
        Sm = jnp.sum(S * m[:, None, :], axis=-1, keepdims=True)
        Sk = jnp.sum(S * k[:, None, :], axis=-1, keepdims=True)
        G = jax.nn.sigmoid(Sm * k[:, None, :] + alpha * S)
        S_new = G * S + (v[:, :, None] - Sk) * k[:, None, :]
        Sq = jnp.sum(S_new * q[:, None, :], axis=-1)
        out_ref[pl.ds(t, 1), :, :] = (Sq * Sq * jax.nn.sigmoid(Sq))[None]
        return S_new

    S_fin = jax.lax.fori_loop(0, _T_CHUNK, body, sf_ref[...])
    sf_ref[...] = S_fin


@jax.jit
def kernel(x, S0, W_kvqm, alpha):
    T, B, D = x.shape
    n = W_kvqm.shape[0] // 4
    wt = W_kvqm.T
    alpha_arr = jnp.reshape(alpha, (1,)).astype(jnp.float32)
    grid = (B // _B_BLK, T // _T_CHUNK)
    out, s_fin = pl.pallas_call(
        _cell_kernel,
        grid=grid,
        in_specs=[
            pl.BlockSpec(memory_space=pltpu.SMEM),
            pl.BlockSpec((_T_CHUNK, _B_BLK, D), lambda b, t: (t, b, 0)),
            pl.BlockSpec((D, 4 * _N), lambda b, t: (0, 0)),
            pl.BlockSpec((_B_BLK, _N, _N), lambda b, t: (b, 0, 0)),
        ],
        out_specs=[
            pl.BlockSpec((_T_CHUNK, _B_BLK, _N), lambda b, t: (t, b, 0)),
            pl.BlockSpec((_B_BLK, _N, _N), lambda b, t: (b, 0, 0)),
        ],
        out_shape=[
            jax.ShapeDtypeStruct((T, B, n), jnp.float32),
            jax.ShapeDtypeStruct((B, n, n), jnp.float32),
        ],
        scratch_shapes=[pltpu.VMEM((_T_CHUNK, _B_BLK, _N), jnp.float32)
                        for _ in range(4)],
        compiler_params=pltpu.CompilerParams(
            dimension_semantics=("parallel", "arbitrary"),
        ),
    )(alpha_arr, x, wt, S0)
    return out, s_fin


# speedup vs baseline: 5.3791x; 1.3994x over previous
import jax
import jax.numpy as jnp
from jax.experimental import pallas as pl
from jax.experimental.pallas import tpu as pltpu

_DIM = 1024
_N = 64
_T_CHUNK = 128
_B_BLK = 16
_EPS_NORM = 1e-6


def _cell_kernel(alpha_ref, x_ref, w_ref, s0_ref, out_ref, sf_ref,
                 k_s, v_s, q_s, m_s):
    t_idx = pl.program_id(1)
    alpha = alpha_ref[0]
    ones_n = jnp.ones((_N, _N), dtype=jnp.float32)

    xc = x_ref[...].reshape(_T_CHUNK * _B_BLK, _DIM)
    proj = jnp.dot(xc, w_ref[...], preferred_element_type=jnp.float32)
    k = proj[:, :_N]
    v = proj[:, _N:2 * _N]
    q = proj[:, 2 * _N:3 * _N]
    m = proj[:, 3 * _N:]
    kk_bc = jnp.dot(k * k, ones_n, preferred_element_type=jnp.float32)
    mm_bc = jnp.dot(m * m, ones_n, preferred_element_type=jnp.float32)
    k = k * (1.0 / (jnp.sqrt(kk_bc) + _EPS_NORM))
    m = m * (1.0 / (jnp.sqrt(mm_bc) + _EPS_NORM))
    k_s[...] = k.reshape(_T_CHUNK, _B_BLK, _N)
    v_s[...] = v.reshape(_T_CHUNK, _B_BLK, _N)
    q_s[...] = q.reshape(_T_CHUNK, _B_BLK, _N)
    m_s[...] = m.reshape(_T_CHUNK, _B_BLK, _N)

    @pl.when(t_idx == 0)
    def _():
        sf_ref[...] = s0_ref[...]

    def body(t, S):
        k = k_s[t][:, None, :]
        q = q_s[t][:, None, :]
        m = m_s[t][:, None, :]
        v = v_s[t][:, :, None]
        Sm_bc = jnp.dot((S * m).reshape(_B_BLK * _N, _N), ones_n,
                        preferred_element_type=jnp.float32
                        ).reshape(_B_BLK, _N, _N)
        Sk_bc = jnp.dot((S * k).reshape(_B_BLK * _N, _N), ones_n,
                        preferred_element_type=jnp.float32
                        ).reshape(_B_BLK, _N, _N)
        G = jax.nn.sigmoid(Sm_bc * k + alpha * S)
        S_new = G * S + (v - Sk_bc) * k
        Sq = jnp.sum(S_new * q, axis=-1)
        out_ref[pl.ds(t, 1), :, :] = Sq[None]
        return S_new

    S_fin = jax.lax.fori_loop(0, _T_CHUNK, body, sf_ref[...])
    sf_ref[...] = S_fin
    Sq_all = out_ref[...]
    out_ref[...] = Sq_all * Sq_all * jax.nn.sigmoid(Sq_all)


@jax.jit
def kernel(x, S0, W_kvqm, alpha):
    T, B, D = x.shape
    n = W_kvqm.shape[0] // 4
    wt = W_kvqm.T
    alpha_arr = jnp.reshape(alpha, (1,)).astype(jnp.float32)
    grid = (B // _B_BLK, T // _T_CHUNK)
    out, s_fin = pl.pallas_call(
        _cell_kernel,
        grid=grid,
        in_specs=[
            pl.BlockSpec(memory_space=pltpu.SMEM),
            pl.BlockSpec((_T_CHUNK, _B_BLK, D), lambda b, t: (t, b, 0)),
            pl.BlockSpec((D, 4 * _N), lambda b, t: (0, 0)),
            pl.BlockSpec((_B_BLK, _N, _N), lambda b, t: (b, 0, 0)),
        ],
        out_specs=[
            pl.BlockSpec((_T_CHUNK, _B_BLK, _N), lambda b, t: (t, b, 0)),
            pl.BlockSpec((_B_BLK, _N, _N), lambda b, t: (b, 0, 0)),
        ],
        out_shape=[
            jax.ShapeDtypeStruct((T, B, n), jnp.float32),
            jax.ShapeDtypeStruct((B, n, n), jnp.float32),
        ],
        scratch_shapes=[pltpu.VMEM((_T_CHUNK, _B_BLK, _N), jnp.float32)
                        for _ in range(4)],
        compiler_params=pltpu.CompilerParams(
            dimension_semantics=("parallel", "arbitrary"),
        ),
    )(alpha_arr, x, wt, S0)
    return out, s_fin
